# baseline (speedup 1.0000x reference)
import functools

import jax
import jax.numpy as jnp
from jax import lax
from jax.experimental import pallas as pl
from jax.experimental.pallas import tpu as pltpu
from jax.experimental.pallas import tpu_sc as plsc

_B, _F, _V, _D = 4096, 26, 1000000, 10
_N = _B * _F
_NA = _N * _D
_NC, _NS = 2, 16
_NW = _NC * _NS
_PER = _NA // _NW
_CH = 128
_NCH = _PER // _CH


def _sc_gather_flat(addr_flat, t_flat):
    mesh = plsc.VectorSubcoreMesh(core_axis_name="c", subcore_axis_name="s")

    @functools.partial(
        pl.kernel,
        out_type=jax.ShapeDtypeStruct((_NA,), jnp.float32),
        mesh=mesh,
        scratch_types=[
            pltpu.VMEM((_PER,), jnp.int32),
            pltpu.VMEM((_PER,), jnp.float32),
            pltpu.SemaphoreType.DMA,
        ],
    )
    def k(addr_hbm, t_hbm, out_hbm, addr_v, val_v, sem):
        w = lax.axis_index("s") * _NC + lax.axis_index("c")
        base = w * _PER
        pltpu.sync_copy(addr_hbm.at[pl.ds(base, _PER)], addr_v)

        @pl.loop(0, _NCH)
        def _(c):
            sl = pl.ds(c * _CH, _CH)
            pltpu.async_copy(t_hbm.at[addr_v.at[sl]], val_v.at[sl], sem)

        @pl.loop(0, _NCH)
        def _(c):
            sl = pl.ds(c * _CH, _CH)
            pltpu.make_async_copy(t_hbm.at[addr_v.at[sl]], val_v.at[sl], sem).wait()

        pltpu.sync_copy(val_v, out_hbm.at[pl.ds(base, _PER)])

    return k(addr_flat, t_flat)


def _towers(gd, gn, dW0, db0, dW1, db1, dW2, db2, dW3, db3,
            nW0, nb0, nW1, nb1, nW2, nb2, nW3, nb3, nW4, nb4):
    def body(gd_ref, gn_ref, dW0r, db0r, dW1r, db1r, dW2r, db2r, dW3r, db3r,
             nW0r, nb0r, nW1r, nb1r, nW2r, nb2r, nW3r, nb3r, nW4r, nb4r,
             defer_o, dnn_o):
        def norm(h):
            mu = jnp.mean(h, axis=0, keepdims=True)
            var = jnp.var(h, axis=0, keepdims=True)
            return (h - mu) / jnp.sqrt(var + 1e-5)

        def mm(a, b):
            return jnp.dot(a, b, preferred_element_type=jnp.float32)

        h = norm(gn_ref[...])
        h = jnp.maximum(mm(h, nW0r[...]) + nb0r[...], 0.0)
        h = jnp.maximum(mm(h, nW1r[...]) + nb1r[...], 0.0)
        h = jnp.maximum(mm(h, nW2r[...]) + nb2r[...], 0.0)
        h = jnp.maximum(mm(h, nW3r[...]) + nb3r[...], 0.0)
        dnn_o[...] = jax.nn.sigmoid(mm(h, nW4r[...]) + nb4r[...])

        g = norm(gd_ref[...])
        g = jnp.maximum(mm(g, dW0r[...]) + db0r[...], 0.0)
        g = jnp.maximum(mm(g, dW1r[...]) + db1r[...], 0.0)
        g = jnp.maximum(mm(g, dW2r[...]) + db2r[...], 0.0)
        defer_o[...] = jax.nn.sigmoid(mm(g, dW3r[...]) + db3r[...])

    out = (jax.ShapeDtypeStruct((_B, 1), jnp.float32),
           jax.ShapeDtypeStruct((_B, 1), jnp.float32))
    return pl.pallas_call(body, out_shape=out)(
        gd, gn, dW0, db0, dW1, db1, dW2, db2, dW3, db3,
        nW0, nb0, nW1, nb1, nW2, nb2, nW3, nb3, nW4, nb4)


def kernel(x, defer_table, dnn_table, dW0, db0, dW1, db1, dW2, db2, dW3, db3,
           nW0, nb0, nW1, nb1, nW2, nb2, nW3, nb3, nW4, nb4):
    addr = (x.reshape(_N, 1).astype(jnp.int32)
            + jnp.arange(_D, dtype=jnp.int32) * _V).reshape(_NA)

    def flat_fm(t):
        return jnp.concatenate([t[:, d] for d in range(_D)], axis=0)

    od = _sc_gather_flat(addr, flat_fm(defer_table))
    on = _sc_gather_flat(addr, flat_fm(dnn_table))
    g_defer = od.reshape(_B, _F * _D)
    g_dnn = on.reshape(_B, _F * _D)
    defer_out, dnn_out = _towers(
        g_defer, g_dnn, dW0, db0, dW1, db1, dW2, db2, dW3, db3,
        nW0, nb0, nW1, nb1, nW2, nb2, nW3, nb3, nW4, nb4)
    return (defer_out, dnn_out)


# speedup vs baseline: 3.7533x; 3.7533x over previous
import functools

import jax
import jax.numpy as jnp
from jax import lax
from jax.experimental import pallas as pl
from jax.experimental.pallas import tpu as pltpu
from jax.experimental.pallas import tpu_sc as plsc

_B, _F, _V, _D = 4096, 26, 1000000, 10
_N = _B * _F
_NA = _N * _D
_NC, _NS = 2, 16
_NW = _NC * _NS
_PER = _NA // _NW
_CH = 128
_NCH = _PER // _CH


def _sc_gather_flat(addr_flat, t_flat):
    mesh = plsc.VectorSubcoreMesh(core_axis_name="c", subcore_axis_name="s")

    @functools.partial(
        pl.kernel,
        out_type=jax.ShapeDtypeStruct((_NA,), jnp.float32),
        mesh=mesh,
        scratch_types=[
            pltpu.VMEM((_PER,), jnp.int32),
            pltpu.VMEM((_PER,), jnp.float32),
            pltpu.SemaphoreType.DMA,
        ],
    )
    def k(addr_hbm, t_hbm, out_hbm, addr_v, val_v, sem):
        w = lax.axis_index("s") * _NC + lax.axis_index("c")
        base = w * _PER
        pltpu.sync_copy(addr_hbm.at[pl.ds(base, _PER)], addr_v)

        @pl.loop(0, _NCH)
        def _(c):
            sl = pl.ds(c * _CH, _CH)
            pltpu.async_copy(t_hbm.at[addr_v.at[sl]], val_v.at[sl], sem)

        @pl.loop(0, _NCH)
        def _(c):
            sl = pl.ds(c * _CH, _CH)
            pltpu.make_async_copy(t_hbm.at[addr_v.at[sl]], val_v.at[sl], sem).wait()

        pltpu.sync_copy(val_v, out_hbm.at[pl.ds(base, _PER)])

    return k(addr_flat, t_flat)


_LB = 16384
_NBLK = (_V + _LB - 1) // _LB
_OBR = _LB * _D // 128
_FLAT = _NBLK * _LB * _D


def _tc_flatten(t):
    tT = t.T

    def body(i_ref, o_ref):
        o_ref[...] = i_ref[...].reshape(_OBR, 128)

    out = pl.pallas_call(
        body,
        grid=(_NBLK,),
        in_specs=[pl.BlockSpec((_D, _LB), lambda i: (0, i))],
        out_specs=pl.BlockSpec((_OBR, 128), lambda i: (i, 0)),
        out_shape=jax.ShapeDtypeStruct((_NBLK * _OBR, 128), jnp.float32),
    )(tT)
    return out.reshape(_FLAT)


def _towers(gd, gn, dW0, db0, dW1, db1, dW2, db2, dW3, db3,
            nW0, nb0, nW1, nb1, nW2, nb2, nW3, nb3, nW4, nb4):
    def body(gd_ref, gn_ref, dW0r, db0r, dW1r, db1r, dW2r, db2r, dW3r, db3r,
             nW0r, nb0r, nW1r, nb1r, nW2r, nb2r, nW3r, nb3r, nW4r, nb4r,
             defer_o, dnn_o):
        def norm(h):
            mu = jnp.mean(h, axis=0, keepdims=True)
            var = jnp.var(h, axis=0, keepdims=True)
            return (h - mu) / jnp.sqrt(var + 1e-5)

        def mm(a, b):
            return jnp.dot(a, b, preferred_element_type=jnp.float32)

        h = norm(gn_ref[...])
        h = jnp.maximum(mm(h, nW0r[...]) + nb0r[...], 0.0)
        h = jnp.maximum(mm(h, nW1r[...]) + nb1r[...], 0.0)
        h = jnp.maximum(mm(h, nW2r[...]) + nb2r[...], 0.0)
        h = jnp.maximum(mm(h, nW3r[...]) + nb3r[...], 0.0)
        dnn_o[...] = jax.nn.sigmoid(mm(h, nW4r[...]) + nb4r[...])

        g = norm(gd_ref[...])
        g = jnp.maximum(mm(g, dW0r[...]) + db0r[...], 0.0)
        g = jnp.maximum(mm(g, dW1r[...]) + db1r[...], 0.0)
        g = jnp.maximum(mm(g, dW2r[...]) + db2r[...], 0.0)
        defer_o[...] = jax.nn.sigmoid(mm(g, dW3r[...]) + db3r[...])

    out = (jax.ShapeDtypeStruct((_B, 1), jnp.float32),
           jax.ShapeDtypeStruct((_B, 1), jnp.float32))
    return pl.pallas_call(body, out_shape=out)(
        gd, gn, dW0, db0, dW1, db1, dW2, db2, dW3, db3,
        nW0, nb0, nW1, nb1, nW2, nb2, nW3, nb3, nW4, nb4)


def kernel(x, defer_table, dnn_table, dW0, db0, dW1, db1, dW2, db2, dW3, db3,
           nW0, nb0, nW1, nb1, nW2, nb2, nW3, nb3, nW4, nb4):
    v = x.reshape(_N, 1).astype(jnp.int32)
    addr = ((v >> 14) * (_LB * _D) + (v & (_LB - 1))
            + jnp.arange(_D, dtype=jnp.int32) * _LB).reshape(_NA)

    od = _sc_gather_flat(addr, _tc_flatten(defer_table))
    on = _sc_gather_flat(addr, _tc_flatten(dnn_table))
    g_defer = od.reshape(_B, _F * _D)
    g_dnn = on.reshape(_B, _F * _D)
    defer_out, dnn_out = _towers(
        g_defer, g_dnn, dW0, db0, dW1, db1, dW2, db2, dW3, db3,
        nW0, nb0, nW1, nb1, nW2, nb2, nW3, nb3, nW4, nb4)
    return (defer_out, dnn_out)


# speedup vs baseline: 3.8523x; 1.0264x over previous
import functools

import jax
import jax.numpy as jnp
from jax import lax
from jax.experimental import pallas as pl
from jax.experimental.pallas import tpu as pltpu
from jax.experimental.pallas import tpu_sc as plsc

_B, _F, _V, _D = 4096, 26, 1000000, 10
_N = _B * _F
_NA = _N * _D
_NC, _NS = 2, 16
_NW = _NC * _NS
_PER = _NA // _NW
_CH = 128
_NCH = _PER // _CH


def _sc_gather_both(addr_flat, t1_flat, t2_flat):
    mesh = plsc.VectorSubcoreMesh(core_axis_name="c", subcore_axis_name="s")

    @functools.partial(
        pl.kernel,
        out_type=(jax.ShapeDtypeStruct((_NA,), jnp.float32),
                  jax.ShapeDtypeStruct((_NA,), jnp.float32)),
        mesh=mesh,
        scratch_types=[
            pltpu.VMEM((_PER,), jnp.int32),
            pltpu.VMEM((_PER,), jnp.float32),
            pltpu.VMEM((_PER,), jnp.float32),
            pltpu.SemaphoreType.DMA,
        ],
    )
    def k(addr_hbm, t1_hbm, t2_hbm, o1_hbm, o2_hbm, addr_v, v1, v2, sem):
        w = lax.axis_index("s") * _NC + lax.axis_index("c")
        base = w * _PER
        pltpu.sync_copy(addr_hbm.at[pl.ds(base, _PER)], addr_v)

        @pl.loop(0, _NCH)
        def _(c):
            sl = pl.ds(c * _CH, _CH)
            pltpu.async_copy(t1_hbm.at[addr_v.at[sl]], v1.at[sl], sem)
            pltpu.async_copy(t2_hbm.at[addr_v.at[sl]], v2.at[sl], sem)

        @pl.loop(0, _NCH)
        def _(c):
            sl = pl.ds(c * _CH, _CH)
            pltpu.make_async_copy(t1_hbm.at[addr_v.at[sl]], v1.at[sl], sem).wait()
            pltpu.make_async_copy(t2_hbm.at[addr_v.at[sl]], v2.at[sl], sem).wait()

        pltpu.sync_copy(v1, o1_hbm.at[pl.ds(base, _PER)])
        pltpu.sync_copy(v2, o2_hbm.at[pl.ds(base, _PER)])

    return k(addr_flat, t1_flat, t2_flat)


_LB = 16384
_NBLK = (_V + _LB - 1) // _LB
_OBR = _LB * _D // 128
_FLAT = _NBLK * _LB * _D


def _tc_flatten2(t1, t2):
    def body(i1_ref, i2_ref, o1_ref, o2_ref):
        o1_ref[...] = i1_ref[...].reshape(_OBR, 128)
        o2_ref[...] = i2_ref[...].reshape(_OBR, 128)

    ispec = pl.BlockSpec((_D, _LB), lambda i: (0, i))
    ospec = pl.BlockSpec((_OBR, 128), lambda i: (i, 0))
    oshape = jax.ShapeDtypeStruct((_NBLK * _OBR, 128), jnp.float32)
    o1, o2 = pl.pallas_call(
        body,
        grid=(_NBLK,),
        in_specs=[ispec, ispec],
        out_specs=[ospec, ospec],
        out_shape=[oshape, oshape],
    )(t1.T, t2.T)
    return o1.reshape(_FLAT), o2.reshape(_FLAT)


def _towers(gd, gn, dW0, db0, dW1, db1, dW2, db2, dW3, db3,
            nW0, nb0, nW1, nb1, nW2, nb2, nW3, nb3, nW4, nb4):
    def body(gd_ref, gn_ref, dW0r, db0r, dW1r, db1r, dW2r, db2r, dW3r, db3r,
             nW0r, nb0r, nW1r, nb1r, nW2r, nb2r, nW3r, nb3r, nW4r, nb4r,
             defer_o, dnn_o):
        def norm(h):
            mu = jnp.mean(h, axis=0, keepdims=True)
            var = jnp.var(h, axis=0, keepdims=True)
            return (h - mu) / jnp.sqrt(var + 1e-5)

        def mm(a, b):
            return jnp.dot(a, b, preferred_element_type=jnp.float32)

        h = norm(gn_ref[...])
        h = jnp.maximum(mm(h, nW0r[...]) + nb0r[...], 0.0)
        h = jnp.maximum(mm(h, nW1r[...]) + nb1r[...], 0.0)
        h = jnp.maximum(mm(h, nW2r[...]) + nb2r[...], 0.0)
        h = jnp.maximum(mm(h, nW3r[...]) + nb3r[...], 0.0)
        dnn_o[...] = jax.nn.sigmoid(mm(h, nW4r[...]) + nb4r[...])

        g = norm(gd_ref[...])
        g = jnp.maximum(mm(g, dW0r[...]) + db0r[...], 0.0)
        g = jnp.maximum(mm(g, dW1r[...]) + db1r[...], 0.0)
        g = jnp.maximum(mm(g, dW2r[...]) + db2r[...], 0.0)
        defer_o[...] = jax.nn.sigmoid(mm(g, dW3r[...]) + db3r[...])

    out = (jax.ShapeDtypeStruct((_B, 1), jnp.float32),
           jax.ShapeDtypeStruct((_B, 1), jnp.float32))
    return pl.pallas_call(body, out_shape=out)(
        gd, gn, dW0, db0, dW1, db1, dW2, db2, dW3, db3,
        nW0, nb0, nW1, nb1, nW2, nb2, nW3, nb3, nW4, nb4)


def kernel(x, defer_table, dnn_table, dW0, db0, dW1, db1, dW2, db2, dW3, db3,
           nW0, nb0, nW1, nb1, nW2, nb2, nW3, nb3, nW4, nb4):
    v = x.reshape(_N, 1).astype(jnp.int32)
    addr = ((v >> 14) * (_LB * _D) + (v & (_LB - 1))
            + jnp.arange(_D, dtype=jnp.int32) * _LB).reshape(_NA)

    td_flat, tn_flat = _tc_flatten2(defer_table, dnn_table)
    od, on = _sc_gather_both(addr, td_flat, tn_flat)
    g_defer = od.reshape(_B, _F * _D)
    g_dnn = on.reshape(_B, _F * _D)
    defer_out, dnn_out = _towers(
        g_defer, g_dnn, dW0, db0, dW1, db1, dW2, db2, dW3, db3,
        nW0, nb0, nW1, nb1, nW2, nb2, nW3, nb3, nW4, nb4)
    return (defer_out, dnn_out)
